# baseline (speedup 1.0000x reference)
import jax
import jax.numpy as jnp
from jax.experimental import pallas as pl
from jax.experimental.pallas import tpu as pltpu

K, B, N, D = 6, 64, 100000, 128
TN = 16768


def _sim_body(pf_ref, mem_ref, out_ref, f16_ref):
    n = pl.program_id(1)

    @pl.when(n == 0)
    def _():
        f = pf_ref[0]
        norm = jnp.sqrt(jnp.sum(f * f, axis=1, keepdims=True))
        f16_ref[...] = (f / jnp.maximum(norm, 1e-12)).astype(jnp.bfloat16)

    m = mem_ref[0].astype(jnp.bfloat16)
    out_ref[0] = jax.lax.dot_general(
        f16_ref[...], m, (((1,), (1,)), ((), ())),
        preferred_element_type=jnp.float32,
    )


def kernel(part_features, memory):
    nb = pl.cdiv(N, TN)
    return pl.pallas_call(
        _sim_body,
        grid=(K, nb),
        in_specs=[
            pl.BlockSpec((1, B, D), lambda k, n: (k, 0, 0)),
            pl.BlockSpec((1, TN, D), lambda k, n: (k, n, 0)),
        ],
        out_specs=pl.BlockSpec((1, B, TN), lambda k, n: (k, 0, n)),
        out_shape=jax.ShapeDtypeStruct((K, B, N), jnp.float32),
        scratch_shapes=[pltpu.VMEM((B, D), jnp.bfloat16)],
        compiler_params=pltpu.CompilerParams(
            dimension_semantics=("parallel", "arbitrary"),
        ),
    )(part_features, memory)


# speedup vs baseline: 1.0382x; 1.0382x over previous
import jax
import jax.numpy as jnp
from jax.experimental import pallas as pl
from jax.experimental.pallas import tpu as pltpu

K, B, N, D = 6, 64, 100000, 128
TN = 33408


def _sim_body(pf_ref, mem_ref, out_ref, f16_ref):
    n = pl.program_id(1)

    @pl.when(n == 0)
    def _():
        f = pf_ref[0]
        norm = jnp.sqrt(jnp.sum(f * f, axis=1, keepdims=True))
        f16_ref[...] = (f / jnp.maximum(norm, 1e-12)).astype(jnp.bfloat16)

    m = mem_ref[0].astype(jnp.bfloat16)
    out_ref[0] = jax.lax.dot_general(
        f16_ref[...], m, (((1,), (1,)), ((), ())),
        preferred_element_type=jnp.float32,
    )


def kernel(part_features, memory):
    nb = pl.cdiv(N, TN)
    return pl.pallas_call(
        _sim_body,
        grid=(K, nb),
        in_specs=[
            pl.BlockSpec((1, B, D), lambda k, n: (k, 0, 0)),
            pl.BlockSpec((1, TN, D), lambda k, n: (k, n, 0)),
        ],
        out_specs=pl.BlockSpec((1, B, TN), lambda k, n: (k, 0, n)),
        out_shape=jax.ShapeDtypeStruct((K, B, N), jnp.float32),
        scratch_shapes=[pltpu.VMEM((B, D), jnp.bfloat16)],
        compiler_params=pltpu.CompilerParams(
            dimension_semantics=("parallel", "arbitrary"),
        ),
    )(part_features, memory)
